# baseline (speedup 1.0000x reference)
import functools

import jax
import jax.numpy as jnp
from jax import lax
from jax.experimental import pallas as pl
from jax.experimental.pallas import tpu as pltpu
from jax.experimental.pallas import tpu_sc as plsc

_NC = 2
_NS = 16
_CH = 80
_NB = 2


def _sc_aggregate(nodes, senders3, receivers3, pad_n):
    n_nodes, d = nodes.shape
    nw, n_ch, ch = senders3.shape
    assert nw == _NC * _NS and ch == _CH
    rows_per_tile = pad_n // _NS
    assert rows_per_tile % 128 == 0

    mesh = plsc.VectorSubcoreMesh(core_axis_name="c", subcore_axis_name="s")

    @functools.partial(
        pl.kernel,
        mesh=mesh,
        out_type=[
            jax.ShapeDtypeStruct((_NC, pad_n, d), jnp.float32),
            jax.ShapeDtypeStruct((_NC, pad_n), jnp.float32),
        ],
        scratch_types=[
            pltpu.VMEM((_NB, _CH), jnp.int32),
            pltpu.VMEM((_NB, _CH), jnp.int32),
            pltpu.VMEM((_CH, d), jnp.float32),
            pltpu.VMEM((_CH, d), jnp.float32),
            pltpu.VMEM((128,), jnp.float32),
            pltpu.VMEM((32, d), jnp.float32),
            pltpu.VMEM((rows_per_tile,), jnp.float32),
            pltpu.VMEM_SHARED((pad_n, d), jnp.float32),
            pltpu.VMEM_SHARED((pad_n,), jnp.float32),
            pltpu.SemaphoreType.DMA,
            pltpu.SemaphoreType.DMA,
            pltpu.SemaphoreType.DMA,
            pltpu.SemaphoreType.DMA,
        ],
    )
    def agg(nodes_h, send_h, recv_h, sum_h, cnt_h,
            sidx, ridx, rows0, rows1, ones, zrows, zcnt,
            acc, cnt,
            rsem0, rsem1, isem0, isem1):
        c = lax.axis_index("c")
        s = lax.axis_index("s")
        wid = c * _NS + s
        rows = (rows0, rows1)
        rsems = (rsem0, rsem1)
        isems = (isem0, isem1)

        def idx_start(j, b):
            pltpu.async_copy(send_h.at[wid, j], sidx.at[b], isems[b])
            pltpu.async_copy(recv_h.at[wid, j], ridx.at[b], isems[b])

        def idx_wait(b):
            pltpu.make_async_copy(send_h.at[wid, 0], sidx.at[b],
                                  isems[b]).wait()
            pltpu.make_async_copy(recv_h.at[wid, 0], ridx.at[b],
                                  isems[b]).wait()

        def gather_start(b):
            pltpu.async_copy(nodes_h.at[sidx.at[b]], rows[b], rsems[b])

        def gather_wait(b):
            pltpu.make_async_copy(nodes_h.at[sidx.at[b]], rows[b],
                                  rsems[b]).wait()

        def scatter(b):
            pltpu.sync_copy(rows[b], acc.at[ridx.at[b]], add=True)
            pltpu.sync_copy(ones.at[pl.ds(0, _CH)], cnt.at[ridx.at[b]],
                            add=True)

        for b in range(_NB):
            idx_start(b, b)

        zero16 = jnp.zeros((16,), jnp.float32)
        one16 = jnp.ones((16,), jnp.float32)

        def zrow_body(i, carry):
            for k in range(d // 16):
                zrows[i, pl.ds(k * 16, 16)] = zero16
            return carry

        lax.fori_loop(0, 32, zrow_body, 0)

        def zcnt_body(i, carry):
            zcnt[pl.ds(i * 16, 16)] = zero16
            return carry

        lax.fori_loop(0, rows_per_tile // 16, zcnt_body, 0)

        for k in range(128 // 16):
            ones[pl.ds(k * 16, 16)] = one16

        r0 = s * rows_per_tile
        for b in range(rows_per_tile // 32):
            pltpu.sync_copy(zrows, acc.at[pl.ds(r0 + b * 32, 32)])
        pltpu.sync_copy(zcnt, cnt.at[pl.ds(r0, rows_per_tile)])
        plsc.subcore_barrier()

        idx_wait(0)
        gather_start(0)

        def group_body(g, carry):
            j0 = _NB * g

            @pl.when(j0 + 1 < n_ch)
            def _():
                idx_wait(1)
                gather_start(1)

            gather_wait(0)
            scatter(0)

            @pl.when(j0 + 2 < n_ch)
            def _():
                idx_start(j0 + 2, 0)
                idx_wait(0)
                gather_start(0)

            @pl.when(j0 + 1 < n_ch)
            def _():
                gather_wait(1)
                scatter(1)

            @pl.when(j0 + 3 < n_ch)
            def _():
                idx_start(j0 + 3, 1)

            return carry

        lax.fori_loop(0, (n_ch + 1) // _NB, group_body, 0)
        plsc.subcore_barrier()

        pltpu.sync_copy(acc.at[pl.ds(r0, rows_per_tile)],
                        sum_h.at[c, pl.ds(r0, rows_per_tile)])
        pltpu.sync_copy(cnt.at[pl.ds(r0, rows_per_tile)],
                        cnt_h.at[c, pl.ds(r0, rows_per_tile)])

    return agg(nodes, senders3, receivers3)


def _tc_dense(nodes_p, w_left, b_left, w_right):
    pn, d = nodes_p.shape
    out_ch = w_right.shape[1]
    blk = 1024
    assert pn % blk == 0

    def body(n_ref, wl_ref, b_ref, wr_ref, o_ref):
        w_comb = wl_ref[0:d, :] + wr_ref[...]
        out = jnp.dot(n_ref[...], w_comb, preferred_element_type=jnp.float32)
        o_ref[...] = out + b_ref[...]

    return pl.pallas_call(
        body,
        grid=(pn // blk,),
        in_specs=[
            pl.BlockSpec((blk, d), lambda i: (i, 0)),
            pl.BlockSpec((2 * d, out_ch), lambda i: (0, 0)),
            pl.BlockSpec((1, out_ch), lambda i: (0, 0)),
            pl.BlockSpec((d, out_ch), lambda i: (0, 0)),
        ],
        out_specs=pl.BlockSpec((blk, out_ch), lambda i: (i, 0)),
        out_shape=jax.ShapeDtypeStruct((pn, out_ch), jnp.float32),
    )(nodes_p, w_left, b_left, w_right)


def _tc_combine(z, s0, s1, c0, c1, w_left):
    pn, d = s0.shape
    out_ch = z.shape[1]
    blk = 1024
    assert pn % blk == 0

    def body(z_ref, s0_ref, s1_ref, c0_ref, c1_ref, wl_ref, o_ref):
        cnt = jnp.maximum(c0_ref[...] + c1_ref[...], 1.0)
        h_agg = (s0_ref[...] + s1_ref[...]) / cnt
        o_ref[...] = z_ref[...] + jnp.dot(
            h_agg, wl_ref[d:, :], preferred_element_type=jnp.float32)

    return pl.pallas_call(
        body,
        grid=(pn // blk,),
        in_specs=[
            pl.BlockSpec((blk, out_ch), lambda i: (i, 0)),
            pl.BlockSpec((blk, d), lambda i: (i, 0)),
            pl.BlockSpec((blk, d), lambda i: (i, 0)),
            pl.BlockSpec((blk, 1), lambda i: (i, 0)),
            pl.BlockSpec((blk, 1), lambda i: (i, 0)),
            pl.BlockSpec((2 * d, d), lambda i: (0, 0)),
        ],
        out_specs=pl.BlockSpec((blk, out_ch), lambda i: (i, 0)),
        out_shape=jax.ShapeDtypeStruct((pn, out_ch), jnp.float32),
    )(z, s0, s1, c0, c1, w_left)


def kernel(nodes, senders, receivers, W_left, b_left, W_right):
    n_nodes, d = nodes.shape
    n_edges = senders.shape[0]
    nw = _NC * _NS
    per_w = n_edges // nw
    assert per_w * nw == n_edges
    per_wp = ((per_w + _CH - 1) // _CH) * _CH
    pad_n = ((n_nodes + 1023) // 1024) * 1024
    senders2 = senders.reshape(nw, per_w)
    receivers2 = receivers.reshape(nw, per_w)
    if per_wp != per_w:
        senders2 = jnp.pad(senders2, ((0, 0), (0, per_wp - per_w)))
        receivers2 = jnp.pad(receivers2, ((0, 0), (0, per_wp - per_w)),
                             constant_values=n_nodes)
    senders3 = senders2.reshape(nw, per_wp // _CH, _CH)
    receivers3 = receivers2.reshape(nw, per_wp // _CH, _CH)
    sums, counts = _sc_aggregate(nodes, senders3, receivers3, pad_n)
    nodes_p = jnp.pad(nodes, ((0, pad_n - n_nodes), (0, 0)))
    z = _tc_dense(nodes_p, W_left, b_left.reshape(1, -1), W_right)
    out = _tc_combine(
        z,
        sums[0], sums[1],
        counts[0].reshape(pad_n, 1), counts[1].reshape(pad_n, 1),
        W_left,
    )
    return out[:n_nodes]


# speedup vs baseline: 1.0341x; 1.0341x over previous
import functools

import jax
import jax.numpy as jnp
from jax import lax
from jax.experimental import pallas as pl
from jax.experimental.pallas import tpu as pltpu
from jax.experimental.pallas import tpu_sc as plsc

_NC = 2
_NS = 16
_CH = 80
_NB = 3


def _sc_aggregate(nodes, senders3, receivers3, pad_n):
    n_nodes, d = nodes.shape
    nw, n_ch, ch = senders3.shape
    assert nw == _NC * _NS and ch == _CH
    rows_per_tile = pad_n // _NS
    assert rows_per_tile % 128 == 0

    mesh = plsc.VectorSubcoreMesh(core_axis_name="c", subcore_axis_name="s")

    @functools.partial(
        pl.kernel,
        mesh=mesh,
        out_type=[
            jax.ShapeDtypeStruct((_NC, pad_n, d), jnp.float32),
            jax.ShapeDtypeStruct((_NC, pad_n), jnp.float32),
        ],
        scratch_types=[
            pltpu.VMEM((_NB, _CH), jnp.int32),
            pltpu.VMEM((_NB, _CH), jnp.int32),
            pltpu.VMEM((_CH, d), jnp.float32),
            pltpu.VMEM((_CH, d), jnp.float32),
            pltpu.VMEM((_CH, d), jnp.float32),
            pltpu.VMEM((128,), jnp.float32),
            pltpu.VMEM((32, d), jnp.float32),
            pltpu.VMEM((rows_per_tile,), jnp.float32),
            pltpu.VMEM_SHARED((pad_n, d), jnp.float32),
            pltpu.VMEM_SHARED((pad_n,), jnp.float32),
            pltpu.SemaphoreType.DMA,
            pltpu.SemaphoreType.DMA,
            pltpu.SemaphoreType.DMA,
            pltpu.SemaphoreType.DMA,
            pltpu.SemaphoreType.DMA,
            pltpu.SemaphoreType.DMA,
        ],
    )
    def agg(nodes_h, send_h, recv_h, sum_h, cnt_h,
            sidx, ridx, rows0, rows1, rows2, ones, zrows, zcnt,
            acc, cnt,
            rsem0, rsem1, rsem2, isem0, isem1, isem2):
        c = lax.axis_index("c")
        s = lax.axis_index("s")
        wid = c * _NS + s
        rows = (rows0, rows1, rows2)
        rsems = (rsem0, rsem1, rsem2)
        isems = (isem0, isem1, isem2)

        def idx_start(j, b):
            pltpu.async_copy(send_h.at[wid, j], sidx.at[b], isems[b])
            pltpu.async_copy(recv_h.at[wid, j], ridx.at[b], isems[b])

        def idx_wait(b):
            pltpu.make_async_copy(send_h.at[wid, 0], sidx.at[b],
                                  isems[b]).wait()
            pltpu.make_async_copy(recv_h.at[wid, 0], ridx.at[b],
                                  isems[b]).wait()

        def gather_start(b):
            pltpu.async_copy(nodes_h.at[sidx.at[b]], rows[b], rsems[b])

        def gather_wait(b):
            pltpu.make_async_copy(nodes_h.at[sidx.at[b]], rows[b],
                                  rsems[b]).wait()

        def scatter(b):
            pltpu.sync_copy(rows[b], acc.at[ridx.at[b]], add=True)
            pltpu.sync_copy(ones.at[pl.ds(0, _CH)], cnt.at[ridx.at[b]],
                            add=True)

        for b in range(_NB):
            idx_start(b, b)

        zero16 = jnp.zeros((16,), jnp.float32)
        one16 = jnp.ones((16,), jnp.float32)

        def zrow_body(i, carry):
            for k in range(d // 16):
                zrows[i, pl.ds(k * 16, 16)] = zero16
            return carry

        lax.fori_loop(0, 32, zrow_body, 0)

        def zcnt_body(i, carry):
            zcnt[pl.ds(i * 16, 16)] = zero16
            return carry

        lax.fori_loop(0, rows_per_tile // 16, zcnt_body, 0)

        for k in range(128 // 16):
            ones[pl.ds(k * 16, 16)] = one16

        r0 = s * rows_per_tile
        for b in range(rows_per_tile // 32):
            pltpu.sync_copy(zrows, acc.at[pl.ds(r0 + b * 32, 32)])
        pltpu.sync_copy(zcnt, cnt.at[pl.ds(r0, rows_per_tile)])
        plsc.subcore_barrier()

        idx_wait(0)
        gather_start(0)

        @pl.when(1 < n_ch)
        def _():
            idx_wait(1)
            gather_start(1)

        def group_body(g, carry):
            j0 = _NB * g
            for b in range(_NB):
                j = j0 + b

                @pl.when(j + 2 < n_ch)
                def _():
                    idx_wait((b + 2) % _NB)
                    gather_start((b + 2) % _NB)

                @pl.when(j < n_ch)
                def _():
                    gather_wait(b)
                    scatter(b)

                @pl.when(j + _NB < n_ch)
                def _():
                    idx_start(j + _NB, b)

            return carry

        lax.fori_loop(0, (n_ch + _NB - 1) // _NB, group_body, 0)
        plsc.subcore_barrier()

        pltpu.sync_copy(acc.at[pl.ds(r0, rows_per_tile)],
                        sum_h.at[c, pl.ds(r0, rows_per_tile)])
        pltpu.sync_copy(cnt.at[pl.ds(r0, rows_per_tile)],
                        cnt_h.at[c, pl.ds(r0, rows_per_tile)])

    return agg(nodes, senders3, receivers3)


def _tc_dense(nodes_p, w_left, b_left, w_right):
    pn, d = nodes_p.shape
    out_ch = w_right.shape[1]
    blk = 1024
    assert pn % blk == 0

    def body(n_ref, wl_ref, b_ref, wr_ref, o_ref):
        w_comb = wl_ref[0:d, :] + wr_ref[...]
        out = jnp.dot(n_ref[...], w_comb, preferred_element_type=jnp.float32)
        o_ref[...] = out + b_ref[...]

    return pl.pallas_call(
        body,
        grid=(pn // blk,),
        in_specs=[
            pl.BlockSpec((blk, d), lambda i: (i, 0)),
            pl.BlockSpec((2 * d, out_ch), lambda i: (0, 0)),
            pl.BlockSpec((1, out_ch), lambda i: (0, 0)),
            pl.BlockSpec((d, out_ch), lambda i: (0, 0)),
        ],
        out_specs=pl.BlockSpec((blk, out_ch), lambda i: (i, 0)),
        out_shape=jax.ShapeDtypeStruct((pn, out_ch), jnp.float32),
    )(nodes_p, w_left, b_left, w_right)


def _tc_combine(z, s0, s1, c0, c1, w_left):
    pn, d = s0.shape
    out_ch = z.shape[1]
    blk = 1024
    assert pn % blk == 0

    def body(z_ref, s0_ref, s1_ref, c0_ref, c1_ref, wl_ref, o_ref):
        cnt = jnp.maximum(c0_ref[...] + c1_ref[...], 1.0)
        h_agg = (s0_ref[...] + s1_ref[...]) / cnt
        o_ref[...] = z_ref[...] + jnp.dot(
            h_agg, wl_ref[d:, :], preferred_element_type=jnp.float32)

    return pl.pallas_call(
        body,
        grid=(pn // blk,),
        in_specs=[
            pl.BlockSpec((blk, out_ch), lambda i: (i, 0)),
            pl.BlockSpec((blk, d), lambda i: (i, 0)),
            pl.BlockSpec((blk, d), lambda i: (i, 0)),
            pl.BlockSpec((blk, 1), lambda i: (i, 0)),
            pl.BlockSpec((blk, 1), lambda i: (i, 0)),
            pl.BlockSpec((2 * d, d), lambda i: (0, 0)),
        ],
        out_specs=pl.BlockSpec((blk, out_ch), lambda i: (i, 0)),
        out_shape=jax.ShapeDtypeStruct((pn, out_ch), jnp.float32),
    )(z, s0, s1, c0, c1, w_left)


def kernel(nodes, senders, receivers, W_left, b_left, W_right):
    n_nodes, d = nodes.shape
    n_edges = senders.shape[0]
    nw = _NC * _NS
    per_w = n_edges // nw
    assert per_w * nw == n_edges
    per_wp = ((per_w + _CH - 1) // _CH) * _CH
    pad_n = ((n_nodes + 1023) // 1024) * 1024
    senders2 = senders.reshape(nw, per_w)
    receivers2 = receivers.reshape(nw, per_w)
    if per_wp != per_w:
        senders2 = jnp.pad(senders2, ((0, 0), (0, per_wp - per_w)))
        receivers2 = jnp.pad(receivers2, ((0, 0), (0, per_wp - per_w)),
                             constant_values=n_nodes)
    senders3 = senders2.reshape(nw, per_wp // _CH, _CH)
    receivers3 = receivers2.reshape(nw, per_wp // _CH, _CH)
    sums, counts = _sc_aggregate(nodes, senders3, receivers3, pad_n)
    nodes_p = jnp.pad(nodes, ((0, pad_n - n_nodes), (0, 0)))
    z = _tc_dense(nodes_p, W_left, b_left.reshape(1, -1), W_right)
    out = _tc_combine(
        z,
        sums[0], sums[1],
        counts[0].reshape(pad_n, 1), counts[1].reshape(pad_n, 1),
        W_left,
    )
    return out[:n_nodes]


# speedup vs baseline: 1.0624x; 1.0275x over previous
import functools

import jax
import jax.numpy as jnp
from jax import lax
from jax.experimental import pallas as pl
from jax.experimental.pallas import tpu as pltpu
from jax.experimental.pallas import tpu_sc as plsc

_NC = 2
_NS = 16
_CH = 80
_NB = 3


def _sc_aggregate(nodes, senders3, receivers3, pad_n):
    n_nodes, d = nodes.shape
    nw, n_ch, ch = senders3.shape
    assert nw == _NC * _NS and ch == _CH
    rows_per_tile = pad_n // _NS
    assert rows_per_tile % 128 == 0

    mesh = plsc.VectorSubcoreMesh(core_axis_name="c", subcore_axis_name="s")

    @functools.partial(
        pl.kernel,
        mesh=mesh,
        out_type=[
            jax.ShapeDtypeStruct((_NC, pad_n, d), jnp.float32),
            jax.ShapeDtypeStruct((_NC, pad_n), jnp.float32),
        ],
        scratch_types=[
            pltpu.VMEM((_NB, _CH), jnp.int32),
            pltpu.VMEM((_NB, _CH), jnp.int32),
            pltpu.VMEM((_CH, d), jnp.float32),
            pltpu.VMEM((_CH, d), jnp.float32),
            pltpu.VMEM((_CH, d), jnp.float32),
            pltpu.VMEM((128,), jnp.float32),
            pltpu.VMEM((32, d), jnp.float32),
            pltpu.VMEM((rows_per_tile,), jnp.float32),
            pltpu.VMEM_SHARED((pad_n, d), jnp.float32),
            pltpu.VMEM_SHARED((pad_n,), jnp.float32),
            pltpu.SemaphoreType.DMA,
            pltpu.SemaphoreType.DMA,
            pltpu.SemaphoreType.DMA,
            pltpu.SemaphoreType.DMA,
            pltpu.SemaphoreType.DMA,
            pltpu.SemaphoreType.DMA,
            pltpu.SemaphoreType.DMA,
            pltpu.SemaphoreType.DMA,
        ],
    )
    def agg(nodes_h, send_h, recv_h, sum_h, cnt_h,
            sidx, ridx, rows0, rows1, rows2, ones, zrows, zcnt,
            acc, cnt,
            rsem0, rsem1, rsem2, isem0, isem1, isem2, ssem, csem):
        c = lax.axis_index("c")
        s = lax.axis_index("s")
        wid = c * _NS + s
        rows = (rows0, rows1, rows2)
        rsems = (rsem0, rsem1, rsem2)
        isems = (isem0, isem1, isem2)

        def idx_start(j, b):
            pltpu.async_copy(send_h.at[wid, j], sidx.at[b], isems[b])
            pltpu.async_copy(recv_h.at[wid, j], ridx.at[b], isems[b])

        def idx_wait(b):
            pltpu.make_async_copy(send_h.at[wid, 0], sidx.at[b],
                                  isems[b]).wait()
            pltpu.make_async_copy(recv_h.at[wid, 0], ridx.at[b],
                                  isems[b]).wait()

        def gather_start(b):
            pltpu.async_copy(nodes_h.at[sidx.at[b]], rows[b], rsems[b])

        def gather_wait(b):
            pltpu.make_async_copy(nodes_h.at[sidx.at[b]], rows[b],
                                  rsems[b]).wait()

        def scatter(b):
            pltpu.async_copy(rows[b], acc.at[ridx.at[b]], ssem, add=True)
            pltpu.async_copy(ones.at[pl.ds(0, _CH)], cnt.at[ridx.at[b]],
                             csem, add=True)
            pltpu.make_async_copy(rows[b], acc.at[ridx.at[b]], ssem).wait()
            pltpu.make_async_copy(ones.at[pl.ds(0, _CH)],
                                  cnt.at[ridx.at[b]], csem).wait()

        for b in range(_NB):
            idx_start(b, b)

        zero16 = jnp.zeros((16,), jnp.float32)
        one16 = jnp.ones((16,), jnp.float32)

        def zrow_body(i, carry):
            for k in range(d // 16):
                zrows[i, pl.ds(k * 16, 16)] = zero16
            return carry

        lax.fori_loop(0, 32, zrow_body, 0)

        def zcnt_body(i, carry):
            zcnt[pl.ds(i * 16, 16)] = zero16
            return carry

        lax.fori_loop(0, rows_per_tile // 16, zcnt_body, 0)

        for k in range(128 // 16):
            ones[pl.ds(k * 16, 16)] = one16

        r0 = s * rows_per_tile
        for b in range(rows_per_tile // 32):
            pltpu.sync_copy(zrows, acc.at[pl.ds(r0 + b * 32, 32)])
        pltpu.sync_copy(zcnt, cnt.at[pl.ds(r0, rows_per_tile)])
        plsc.subcore_barrier()

        idx_wait(0)
        gather_start(0)

        @pl.when(1 < n_ch)
        def _():
            idx_wait(1)
            gather_start(1)

        def group_body(g, carry):
            j0 = _NB * g
            for b in range(_NB):
                j = j0 + b

                @pl.when(j + 2 < n_ch)
                def _():
                    idx_wait((b + 2) % _NB)
                    gather_start((b + 2) % _NB)

                @pl.when(j < n_ch)
                def _():
                    gather_wait(b)
                    scatter(b)

                @pl.when(j + _NB < n_ch)
                def _():
                    idx_start(j + _NB, b)

            return carry

        lax.fori_loop(0, (n_ch + _NB - 1) // _NB, group_body, 0)
        plsc.subcore_barrier()

        pltpu.sync_copy(acc.at[pl.ds(r0, rows_per_tile)],
                        sum_h.at[c, pl.ds(r0, rows_per_tile)])
        pltpu.sync_copy(cnt.at[pl.ds(r0, rows_per_tile)],
                        cnt_h.at[c, pl.ds(r0, rows_per_tile)])

    return agg(nodes, senders3, receivers3)


def _tc_dense(nodes_p, w_left, b_left, w_right):
    pn, d = nodes_p.shape
    out_ch = w_right.shape[1]
    blk = 1024
    assert pn % blk == 0

    def body(n_ref, wl_ref, b_ref, wr_ref, o_ref):
        w_comb = wl_ref[0:d, :] + wr_ref[...]
        out = jnp.dot(n_ref[...], w_comb, preferred_element_type=jnp.float32)
        o_ref[...] = out + b_ref[...]

    return pl.pallas_call(
        body,
        grid=(pn // blk,),
        in_specs=[
            pl.BlockSpec((blk, d), lambda i: (i, 0)),
            pl.BlockSpec((2 * d, out_ch), lambda i: (0, 0)),
            pl.BlockSpec((1, out_ch), lambda i: (0, 0)),
            pl.BlockSpec((d, out_ch), lambda i: (0, 0)),
        ],
        out_specs=pl.BlockSpec((blk, out_ch), lambda i: (i, 0)),
        out_shape=jax.ShapeDtypeStruct((pn, out_ch), jnp.float32),
    )(nodes_p, w_left, b_left, w_right)


def _tc_combine(z, s0, s1, c0, c1, w_left):
    pn, d = s0.shape
    out_ch = z.shape[1]
    blk = 1024
    assert pn % blk == 0

    def body(z_ref, s0_ref, s1_ref, c0_ref, c1_ref, wl_ref, o_ref):
        cnt = jnp.maximum(c0_ref[...] + c1_ref[...], 1.0)
        h_agg = (s0_ref[...] + s1_ref[...]) / cnt
        o_ref[...] = z_ref[...] + jnp.dot(
            h_agg, wl_ref[d:, :], preferred_element_type=jnp.float32)

    return pl.pallas_call(
        body,
        grid=(pn // blk,),
        in_specs=[
            pl.BlockSpec((blk, out_ch), lambda i: (i, 0)),
            pl.BlockSpec((blk, d), lambda i: (i, 0)),
            pl.BlockSpec((blk, d), lambda i: (i, 0)),
            pl.BlockSpec((blk, 1), lambda i: (i, 0)),
            pl.BlockSpec((blk, 1), lambda i: (i, 0)),
            pl.BlockSpec((2 * d, d), lambda i: (0, 0)),
        ],
        out_specs=pl.BlockSpec((blk, out_ch), lambda i: (i, 0)),
        out_shape=jax.ShapeDtypeStruct((pn, out_ch), jnp.float32),
    )(z, s0, s1, c0, c1, w_left)


def kernel(nodes, senders, receivers, W_left, b_left, W_right):
    n_nodes, d = nodes.shape
    n_edges = senders.shape[0]
    nw = _NC * _NS
    per_w = n_edges // nw
    assert per_w * nw == n_edges
    per_wp = ((per_w + _CH - 1) // _CH) * _CH
    pad_n = ((n_nodes + 1023) // 1024) * 1024
    senders2 = senders.reshape(nw, per_w)
    receivers2 = receivers.reshape(nw, per_w)
    if per_wp != per_w:
        senders2 = jnp.pad(senders2, ((0, 0), (0, per_wp - per_w)))
        receivers2 = jnp.pad(receivers2, ((0, 0), (0, per_wp - per_w)),
                             constant_values=n_nodes)
    senders3 = senders2.reshape(nw, per_wp // _CH, _CH)
    receivers3 = receivers2.reshape(nw, per_wp // _CH, _CH)
    sums, counts = _sc_aggregate(nodes, senders3, receivers3, pad_n)
    nodes_p = jnp.pad(nodes, ((0, pad_n - n_nodes), (0, 0)))
    z = _tc_dense(nodes_p, W_left, b_left.reshape(1, -1), W_right)
    out = _tc_combine(
        z,
        sums[0], sums[1],
        counts[0].reshape(pad_n, 1), counts[1].reshape(pad_n, 1),
        W_left,
    )
    return out[:n_nodes]


# speedup vs baseline: 1.2068x; 1.1358x over previous
import functools

import jax
import jax.numpy as jnp
from jax import lax
from jax.experimental import pallas as pl
from jax.experimental.pallas import tpu as pltpu
from jax.experimental.pallas import tpu_sc as plsc

_NC = 2
_NS = 16
_CH = 80
_NB = 3


def _sc_aggregate(nodes, senders, receivers, pad_n, per_w):
    n_nodes, d = nodes.shape
    assert per_w % _CH == 0 and (per_w * _NC * _NS,) == senders.shape
    n_ch = per_w // _CH
    rows_per_tile = pad_n // _NS
    assert rows_per_tile % 128 == 0

    mesh = plsc.VectorSubcoreMesh(core_axis_name="c", subcore_axis_name="s")

    @functools.partial(
        pl.kernel,
        mesh=mesh,
        out_type=[
            jax.ShapeDtypeStruct((_NC, pad_n, d), jnp.float32),
            jax.ShapeDtypeStruct((_NC, pad_n), jnp.float32),
        ],
        scratch_types=[
            pltpu.VMEM((_NB, _CH), jnp.int32),
            pltpu.VMEM((_NB, _CH), jnp.int32),
            pltpu.VMEM((_CH, d), jnp.float32),
            pltpu.VMEM((_CH, d), jnp.float32),
            pltpu.VMEM((_CH, d), jnp.float32),
            pltpu.VMEM((128,), jnp.float32),
            pltpu.VMEM((32, d), jnp.float32),
            pltpu.VMEM((rows_per_tile,), jnp.float32),
            pltpu.VMEM_SHARED((pad_n, d), jnp.float32),
            pltpu.VMEM_SHARED((pad_n,), jnp.float32),
            pltpu.SemaphoreType.DMA,
            pltpu.SemaphoreType.DMA,
            pltpu.SemaphoreType.DMA,
            pltpu.SemaphoreType.DMA,
            pltpu.SemaphoreType.DMA,
            pltpu.SemaphoreType.DMA,
            pltpu.SemaphoreType.DMA,
            pltpu.SemaphoreType.DMA,
        ],
    )
    def agg(nodes_h, send_h, recv_h, sum_h, cnt_h,
            sidx, ridx, rows0, rows1, rows2, ones, zrows, zcnt,
            acc, cnt,
            rsem0, rsem1, rsem2, isem0, isem1, isem2, ssem, csem):
        c = lax.axis_index("c")
        s = lax.axis_index("s")
        wid = c * _NS + s
        rows = (rows0, rows1, rows2)
        rsems = (rsem0, rsem1, rsem2)
        isems = (isem0, isem1, isem2)

        def idx_start(j, b):
            off = wid * per_w + j * _CH
            pltpu.async_copy(send_h.at[pl.ds(off, _CH)], sidx.at[b],
                             isems[b])
            pltpu.async_copy(recv_h.at[pl.ds(off, _CH)], ridx.at[b],
                             isems[b])

        def idx_wait(b):
            pltpu.make_async_copy(send_h.at[pl.ds(0, _CH)], sidx.at[b],
                                  isems[b]).wait()
            pltpu.make_async_copy(recv_h.at[pl.ds(0, _CH)], ridx.at[b],
                                  isems[b]).wait()

        def gather_start(b):
            pltpu.async_copy(nodes_h.at[sidx.at[b]], rows[b], rsems[b])

        def gather_wait(b):
            pltpu.make_async_copy(nodes_h.at[sidx.at[b]], rows[b],
                                  rsems[b]).wait()

        def scatter(b):
            pltpu.async_copy(rows[b], acc.at[ridx.at[b]], ssem, add=True)
            pltpu.async_copy(ones.at[pl.ds(0, _CH)], cnt.at[ridx.at[b]],
                             csem, add=True)
            pltpu.make_async_copy(rows[b], acc.at[ridx.at[b]], ssem).wait()
            pltpu.make_async_copy(ones.at[pl.ds(0, _CH)],
                                  cnt.at[ridx.at[b]], csem).wait()

        for b in range(_NB):
            idx_start(b, b)

        zero16 = jnp.zeros((16,), jnp.float32)
        one16 = jnp.ones((16,), jnp.float32)

        def zrow_body(i, carry):
            for k in range(d // 16):
                zrows[i, pl.ds(k * 16, 16)] = zero16
            return carry

        lax.fori_loop(0, 32, zrow_body, 0)

        def zcnt_body(i, carry):
            zcnt[pl.ds(i * 16, 16)] = zero16
            return carry

        lax.fori_loop(0, rows_per_tile // 16, zcnt_body, 0)

        for k in range(128 // 16):
            ones[pl.ds(k * 16, 16)] = one16

        r0 = s * rows_per_tile
        for b in range(rows_per_tile // 32):
            pltpu.sync_copy(zrows, acc.at[pl.ds(r0 + b * 32, 32)])
        pltpu.sync_copy(zcnt, cnt.at[pl.ds(r0, rows_per_tile)])
        plsc.subcore_barrier()

        idx_wait(0)
        gather_start(0)

        @pl.when(1 < n_ch)
        def _():
            idx_wait(1)
            gather_start(1)

        def group_body(g, carry):
            j0 = _NB * g
            for b in range(_NB):
                j = j0 + b

                @pl.when(j + 2 < n_ch)
                def _():
                    idx_wait((b + 2) % _NB)
                    gather_start((b + 2) % _NB)

                @pl.when(j < n_ch)
                def _():
                    gather_wait(b)
                    scatter(b)

                @pl.when(j + _NB < n_ch)
                def _():
                    idx_start(j + _NB, b)

            return carry

        lax.fori_loop(0, (n_ch + _NB - 1) // _NB, group_body, 0)
        plsc.subcore_barrier()

        pltpu.sync_copy(acc.at[pl.ds(r0, rows_per_tile)],
                        sum_h.at[c, pl.ds(r0, rows_per_tile)])
        pltpu.sync_copy(cnt.at[pl.ds(r0, rows_per_tile)],
                        cnt_h.at[c, pl.ds(r0, rows_per_tile)])

    return agg(nodes, senders, receivers)


def _tc_dense(nodes, w_left, b_left, w_right):
    n, d = nodes.shape
    out_ch = w_right.shape[1]
    blk = 2000
    assert n % blk == 0

    def body(n_ref, wl_ref, b_ref, wr_ref, o_ref):
        w_comb = wl_ref[0:d, :] + wr_ref[...]
        out = jnp.dot(n_ref[...], w_comb, preferred_element_type=jnp.float32)
        o_ref[...] = out + b_ref[...]

    return pl.pallas_call(
        body,
        grid=(n // blk,),
        in_specs=[
            pl.BlockSpec((blk, d), lambda i: (i, 0)),
            pl.BlockSpec((2 * d, out_ch), lambda i: (0, 0)),
            pl.BlockSpec((1, out_ch), lambda i: (0, 0)),
            pl.BlockSpec((d, out_ch), lambda i: (0, 0)),
        ],
        out_specs=pl.BlockSpec((blk, out_ch), lambda i: (i, 0)),
        out_shape=jax.ShapeDtypeStruct((n, out_ch), jnp.float32),
    )(nodes, w_left, b_left, w_right)


def _tc_combine(z, sums, counts, w_left):
    n, out_ch = z.shape
    d = sums.shape[2]
    blk = 2000
    assert n % blk == 0

    def body(z_ref, s_ref, c_ref, wl_ref, o_ref):
        cnt = jnp.maximum(c_ref[0] + c_ref[1], 1.0)
        h_agg = (s_ref[0] + s_ref[1]) / cnt
        o_ref[...] = z_ref[...] + jnp.dot(
            h_agg, wl_ref[d:, :], preferred_element_type=jnp.float32)

    return pl.pallas_call(
        body,
        grid=(n // blk,),
        in_specs=[
            pl.BlockSpec((blk, out_ch), lambda i: (i, 0)),
            pl.BlockSpec((_NC, blk, d), lambda i: (0, i, 0)),
            pl.BlockSpec((_NC, blk, 1), lambda i: (0, i, 0)),
            pl.BlockSpec((2 * d, out_ch), lambda i: (0, 0)),
        ],
        out_specs=pl.BlockSpec((blk, out_ch), lambda i: (i, 0)),
        out_shape=jax.ShapeDtypeStruct((n, out_ch), jnp.float32),
    )(z, sums, counts, w_left)


def kernel(nodes, senders, receivers, W_left, b_left, W_right):
    n_nodes, d = nodes.shape
    n_edges = senders.shape[0]
    nw = _NC * _NS
    per_w = n_edges // nw
    assert per_w * nw == n_edges and per_w % _CH == 0
    pad_n = ((n_nodes + 1023) // 1024) * 1024
    sums, counts = _sc_aggregate(nodes, senders, receivers, pad_n, per_w)
    z = _tc_dense(nodes, W_left, b_left.reshape(1, -1), W_right)
    return _tc_combine(z, sums, counts.reshape(_NC, pad_n, 1), W_left)


# speedup vs baseline: 1.5847x; 1.3132x over previous
import functools

import jax
import jax.numpy as jnp
from jax import lax
from jax.experimental import pallas as pl
from jax.experimental.pallas import tpu as pltpu
from jax.experimental.pallas import tpu_sc as plsc

_NC = 2
_NS = 16
_CH = 80
_NB = 3
_NI = 6


def _sc_aggregate(nodes, senders, receivers, pad_n, per_w):
    n_nodes, d = nodes.shape
    assert per_w % _CH == 0 and (per_w * _NC * _NS,) == senders.shape
    n_ch = per_w // _CH
    rows_per_tile = pad_n // _NS
    assert rows_per_tile % 128 == 0

    mesh = plsc.VectorSubcoreMesh(core_axis_name="c", subcore_axis_name="s")

    @functools.partial(
        pl.kernel,
        mesh=mesh,
        out_type=[
            jax.ShapeDtypeStruct((_NC, pad_n, d), jnp.float32),
            jax.ShapeDtypeStruct((_NC, pad_n), jnp.float32),
        ],
        scratch_types=[
            pltpu.VMEM((_NI, _CH), jnp.int32),
            pltpu.VMEM((_NI, _CH), jnp.int32),
            pltpu.VMEM((_CH, d), jnp.float32),
            pltpu.VMEM((_CH, d), jnp.float32),
            pltpu.VMEM((_CH, d), jnp.float32),
            pltpu.VMEM((128,), jnp.float32),
            pltpu.VMEM((32, d), jnp.float32),
            pltpu.VMEM((rows_per_tile,), jnp.float32),
            pltpu.VMEM_SHARED((pad_n, d), jnp.float32),
            pltpu.VMEM_SHARED((pad_n,), jnp.float32),
            pltpu.SemaphoreType.DMA,
            pltpu.SemaphoreType.DMA,
            pltpu.SemaphoreType.DMA,
            pltpu.SemaphoreType.DMA,
            pltpu.SemaphoreType.DMA,
            pltpu.SemaphoreType.DMA,
            pltpu.SemaphoreType.DMA,
            pltpu.SemaphoreType.DMA,
        ],
    )
    def agg(nodes_h, send_h, recv_h, sum_h, cnt_h,
            sidx, ridx, rows0, rows1, rows2, ones, zrows, zcnt,
            acc, cnt,
            rsem0, rsem1, rsem2, isem0, isem1, isem2, ssem, csem):
        c = lax.axis_index("c")
        s = lax.axis_index("s")
        wid = c * _NS + s
        rows = (rows0, rows1, rows2)
        rsems = (rsem0, rsem1, rsem2)
        isems = (isem0, isem1, isem2)

        def idx_start(j, bi):
            off = wid * per_w + j * _CH
            pltpu.async_copy(send_h.at[pl.ds(off, _CH)], sidx.at[bi],
                             isems[bi % _NB])
            pltpu.async_copy(recv_h.at[pl.ds(off, _CH)], ridx.at[bi],
                             isems[bi % _NB])

        def idx_wait(bi):
            pltpu.make_async_copy(send_h.at[pl.ds(0, _CH)], sidx.at[bi],
                                  isems[bi % _NB]).wait()
            pltpu.make_async_copy(recv_h.at[pl.ds(0, _CH)], ridx.at[bi],
                                  isems[bi % _NB]).wait()

        def gather_start(bi, br):
            pltpu.async_copy(nodes_h.at[sidx.at[bi]], rows[br], rsems[br])

        def gather_wait(bi, br):
            pltpu.make_async_copy(nodes_h.at[sidx.at[bi]], rows[br],
                                  rsems[br]).wait()

        def scatter_issue(bi, br):
            pltpu.async_copy(rows[br], acc.at[ridx.at[bi]], ssem, add=True)
            pltpu.async_copy(ones.at[pl.ds(0, _CH)], cnt.at[ridx.at[bi]],
                             csem, add=True)

        def scatter_wait(bi, br):
            pltpu.make_async_copy(rows[br], acc.at[ridx.at[bi]],
                                  ssem).wait()
            pltpu.make_async_copy(ones.at[pl.ds(0, _CH)],
                                  cnt.at[ridx.at[bi]], csem).wait()

        for b in range(_NI - 1):
            idx_start(b, b)

        zero16 = jnp.zeros((16,), jnp.float32)
        one16 = jnp.ones((16,), jnp.float32)

        def zrow_body(i, carry):
            for k in range(d // 16):
                zrows[i, pl.ds(k * 16, 16)] = zero16
            return carry

        lax.fori_loop(0, 32, zrow_body, 0)

        def zcnt_body(i, carry):
            zcnt[pl.ds(i * 16, 16)] = zero16
            return carry

        lax.fori_loop(0, rows_per_tile // 16, zcnt_body, 0)

        for k in range(128 // 16):
            ones[pl.ds(k * 16, 16)] = one16

        r0 = s * rows_per_tile
        for b in range(rows_per_tile // 32):
            pltpu.sync_copy(zrows, acc.at[pl.ds(r0 + b * 32, 32)])
        pltpu.sync_copy(zcnt, cnt.at[pl.ds(r0, rows_per_tile)])
        plsc.subcore_barrier()

        idx_wait(0)
        gather_start(0, 0)

        @pl.when(1 < n_ch)
        def _():
            idx_wait(1)
            gather_start(1, 1)

        def group_body(g, carry):
            j0 = _NI * g
            for u in range(_NI):
                j = j0 + u

                @pl.when(jnp.logical_and(j >= 1, j <= n_ch))
                def _():
                    scatter_wait((u - 1) % _NI, (u - 1) % _NB)

                @pl.when(j + 5 < n_ch)
                def _():
                    idx_start(j + 5, (u + 5) % _NI)

                @pl.when(j + 2 < n_ch)
                def _():
                    idx_wait((u + 2) % _NI)
                    gather_start((u + 2) % _NI, (u + 2) % _NB)

                @pl.when(j < n_ch)
                def _():
                    gather_wait(u, u % _NB)
                    scatter_issue(u, u % _NB)

            return carry

        lax.fori_loop(0, (n_ch + _NI) // _NI, group_body, 0)
        plsc.subcore_barrier()

        pltpu.sync_copy(acc.at[pl.ds(r0, rows_per_tile)],
                        sum_h.at[c, pl.ds(r0, rows_per_tile)])
        pltpu.sync_copy(cnt.at[pl.ds(r0, rows_per_tile)],
                        cnt_h.at[c, pl.ds(r0, rows_per_tile)])

    return agg(nodes, senders, receivers)


def _tc_dense(nodes, w_left, b_left, w_right):
    n, d = nodes.shape
    out_ch = w_right.shape[1]
    blk = 2000
    assert n % blk == 0

    def body(n_ref, wl_ref, b_ref, wr_ref, o_ref):
        w_comb = wl_ref[0:d, :] + wr_ref[...]
        out = jnp.dot(n_ref[...], w_comb, preferred_element_type=jnp.float32)
        o_ref[...] = out + b_ref[...]

    return pl.pallas_call(
        body,
        grid=(n // blk,),
        in_specs=[
            pl.BlockSpec((blk, d), lambda i: (i, 0)),
            pl.BlockSpec((2 * d, out_ch), lambda i: (0, 0)),
            pl.BlockSpec((1, out_ch), lambda i: (0, 0)),
            pl.BlockSpec((d, out_ch), lambda i: (0, 0)),
        ],
        out_specs=pl.BlockSpec((blk, out_ch), lambda i: (i, 0)),
        out_shape=jax.ShapeDtypeStruct((n, out_ch), jnp.float32),
    )(nodes, w_left, b_left, w_right)


def _tc_combine(z, sums, counts, w_left):
    n, out_ch = z.shape
    d = sums.shape[2]
    blk = 2000
    assert n % blk == 0

    def body(z_ref, s_ref, c_ref, wl_ref, o_ref):
        cnt = jnp.maximum(c_ref[0] + c_ref[1], 1.0)
        h_agg = (s_ref[0] + s_ref[1]) / cnt
        o_ref[...] = z_ref[...] + jnp.dot(
            h_agg, wl_ref[d:, :], preferred_element_type=jnp.float32)

    return pl.pallas_call(
        body,
        grid=(n // blk,),
        in_specs=[
            pl.BlockSpec((blk, out_ch), lambda i: (i, 0)),
            pl.BlockSpec((_NC, blk, d), lambda i: (0, i, 0)),
            pl.BlockSpec((_NC, blk, 1), lambda i: (0, i, 0)),
            pl.BlockSpec((2 * d, out_ch), lambda i: (0, 0)),
        ],
        out_specs=pl.BlockSpec((blk, out_ch), lambda i: (i, 0)),
        out_shape=jax.ShapeDtypeStruct((n, out_ch), jnp.float32),
    )(z, sums, counts, w_left)


def kernel(nodes, senders, receivers, W_left, b_left, W_right):
    n_nodes, d = nodes.shape
    n_edges = senders.shape[0]
    nw = _NC * _NS
    per_w = n_edges // nw
    assert per_w * nw == n_edges and per_w % _CH == 0
    pad_n = ((n_nodes + 1023) // 1024) * 1024
    sums, counts = _sc_aggregate(nodes, senders, receivers, pad_n, per_w)
    z = _tc_dense(nodes, W_left, b_left.reshape(1, -1), W_right)
    return _tc_combine(z, sums, counts.reshape(_NC, pad_n, 1), W_left)
